# baseline (speedup 1.0000x reference)
import functools

import jax
import jax.numpy as jnp
from jax import lax
from jax.experimental import pallas as pl
from jax.experimental.pallas import tpu as pltpu
from jax.experimental.pallas import tpu_sc as plsc

B = 16384
D = 64
D2 = 2 * D
NC = 2
NS = 16
NW = NC * NS
BPW = B // NW
HBUF = 256
CHUNK = 128
NCK = BPW // CHUNK

CB = 16384


def _stage1_body(gu_t, gi_t, mu_t, mi_t, wu, wi, u_o, i_o):
    dn = (((0,), (0,)), ((), ()))

    def two(a_t, b_t, w):
        x = jnp.concatenate([a_t[...], b_t[...]], axis=0)
        full = lax.dot_general(x.astype(jnp.bfloat16), w[...],
                               dimension_numbers=dn,
                               preferred_element_type=jnp.float32)
        return pltpu.bitcast(full.astype(jnp.bfloat16), jnp.float32)

    u_o[...] = two(gu_t, mu_t, wu)
    i_o[...] = two(gi_t, mi_t, wi)


def _stage1(gu_t, gi_t, mu_t, mi_t, wu, wi):
    n = gu_t.shape[1]
    col_spec = pl.BlockSpec((D, CB), lambda i: (0, i))
    w_spec = pl.BlockSpec((D2, D2), lambda i: (0, 0))
    out_spec = pl.BlockSpec((CB // 2, D2), lambda i: (i, 0))
    return pl.pallas_call(
        _stage1_body,
        grid=(pl.cdiv(n, CB),),
        in_specs=[col_spec] * 4 + [w_spec] * 2,
        out_specs=[out_spec] * 2,
        out_shape=[jax.ShapeDtypeStruct((n // 2, D2), jnp.float32)] * 2,
    )(gu_t, gi_t, mu_t, mi_t, wu, wi)


def _sc_gather_body(u_tab, i_tab, uidx, iidx, u_out, i_out,
                    uidx_v, iidx_v, buf_a, buf_b, sem_a, sem_b):
    wid = lax.axis_index("s") * NC + lax.axis_index("c")
    base = wid * BPW
    row = wid * NCK
    pltpu.sync_copy(uidx.at[pl.ds(row, NCK)], uidx_v)
    pltpu.sync_copy(iidx.at[pl.ds(row, NCK)], iidx_v)

    for h in range(BPW // HBUF):
        cps = []
        for j in range(HBUF // CHUNK):
            c = h * (HBUF // CHUNK) + j
            cps.append(pltpu.async_copy(
                u_tab.at[uidx_v.at[c]],
                buf_a.at[pl.ds(j * CHUNK, CHUNK)], sem_a))
            cps.append(pltpu.async_copy(
                i_tab.at[iidx_v.at[c]],
                buf_b.at[pl.ds(j * CHUNK, CHUNK)], sem_b))
        for cp in cps:
            cp.wait()
        pltpu.sync_copy(buf_a, u_out.at[pl.ds(base + h * HBUF, HBUF)])
        pltpu.sync_copy(buf_b, i_out.at[pl.ds(base + h * HBUF, HBUF)])


def _sc_gather(u_tab, i_tab, uidx, iidx):
    mesh = plsc.VectorSubcoreMesh(core_axis_name="c", subcore_axis_name="s")
    run = functools.partial(
        pl.kernel,
        out_type=[jax.ShapeDtypeStruct((B, D2), jnp.float32)] * 2,
        mesh=mesh,
        scratch_types=[
            pltpu.VMEM((NCK, CHUNK), jnp.int32),
            pltpu.VMEM((NCK, CHUNK), jnp.int32),
            pltpu.VMEM((HBUF, D2), jnp.float32),
            pltpu.VMEM((HBUF, D2), jnp.float32),
            pltpu.SemaphoreType.DMA,
            pltpu.SemaphoreType.DMA,
        ],
    )(_sc_gather_body)
    return run(u_tab, i_tab, uidx, iidx)


TILE = 2048


def _dense_body(u_r, i_r, pu, pi, b1, w2, b2, w3, b3, wo2, bo, out):
    def unpack(packed, parity):
        w = lax.bitcast_convert_type(packed, jnp.uint32)
        even = lax.bitcast_convert_type(w << 16, jnp.float32)
        odd = lax.bitcast_convert_type(w & jnp.uint32(0xFFFF0000), jnp.float32)
        return jnp.where(parity > 0.5, odd, even)

    u = unpack(u_r[...], pu[...])
    i = unpack(i_r[...], pi[...])
    gu = u[:, :D]
    mu = u[:, D:]
    gi = i[:, :D]
    mi = i[:, D:]
    h = jnp.maximum(mu + mi + b1[...], 0.0)
    h = jnp.maximum(
        jnp.dot(h, w2[...], preferred_element_type=jnp.float32) + b2[...], 0.0)
    h = jnp.maximum(
        jnp.dot(h, w3[...], preferred_element_type=jnp.float32) + b3[...], 0.0)
    logit = (jnp.sum(gu * gi, axis=1, keepdims=True)
             + jnp.sum(h * wo2[...], axis=1, keepdims=True) + bo[...])
    out[...] = 1.0 / (1.0 + jnp.exp(-logit))


def _dense(u_r, i_r, pu, pi, b1, w2, b2, w3, b3, wo2, bo):
    row_spec = pl.BlockSpec((TILE, D2), lambda i: (i, 0))
    par_spec = pl.BlockSpec((TILE, 1), lambda i: (i, 0))
    full = lambda shape: pl.BlockSpec(shape, lambda i: (0, 0))
    return pl.pallas_call(
        _dense_body,
        grid=(B // TILE,),
        in_specs=[
            row_spec, row_spec, par_spec, par_spec,
            full((1, 64)),
            full((64, 32)), full((1, 32)),
            full((32, 16)), full((1, 16)),
            full((1, 16)), full((1, 1)),
        ],
        out_specs=pl.BlockSpec((TILE, 1), lambda i: (i, 0)),
        out_shape=jax.ShapeDtypeStruct((B, 1), jnp.float32),
    )(u_r, i_r, pu, pi, b1, w2, b2, w3, b3, wo2, bo)


def kernel(user_input, item_input, gmf_user, gmf_item, mlp_user, mlp_item,
           W1, b1, W2, b2, W3, b3, Wo, bo):
    ui32 = user_input.astype(jnp.int32)
    ii32 = item_input.astype(jnp.int32)
    uidx = (ui32 >> 1).reshape(B // CHUNK, CHUNK)
    iidx = (ii32 >> 1).reshape(B // CHUNK, CHUNK)
    pu = (ui32 & 1).astype(jnp.float32).reshape(B, 1)
    pi = (ii32 & 1).astype(jnp.float32).reshape(B, 1)

    ones = jnp.ones((), jnp.float32)
    diag_wo = jnp.diag(Wo[:D, 0])
    diag_one = jnp.diag(jnp.broadcast_to(ones, (D,)))
    z = jnp.zeros((D, D), jnp.float32)
    wu = jnp.block([[diag_wo, z], [z, W1[:D]]])
    wi = jnp.block([[diag_one, z], [z, W1[D:]]])
    u_tab, i_tab = _stage1(
        gmf_user.T, gmf_item.T, mlp_user.T, mlp_item.T,
        wu.astype(jnp.bfloat16), wi.astype(jnp.bfloat16))
    u_rows, i_rows = _sc_gather(u_tab, i_tab, uidx, iidx)

    return _dense(
        u_rows, i_rows, pu, pi,
        b1.reshape(1, 64), W2, b2.reshape(1, 32), W3, b3.reshape(1, 16),
        Wo[D:, 0].reshape(1, 16), bo.reshape(1, 1))
